# baseline (speedup 1.0000x reference)
import functools

import jax
import jax.numpy as jnp
from jax import lax
from jax.experimental import pallas as pl
from jax.experimental.pallas import tpu as pltpu
from jax.experimental.pallas import tpu_sc as plsc

NC = 2
NS = 16
NW = NC * NS
C = 128
PAD_ROWS = 112


def _fill_const(ref, rows, d, val):
    @pl.loop(0, rows)
    def _(r):
        for cc in range(d // 16):
            ref[r, pl.ds(cc * 16, 16)] = jnp.full((16,), val, jnp.float32)


def _make_agg(n_rows, d, k_chunks, gather, nbuf, c=C, s_chk=1, n_phases=1,
              tc_tiling=True):
    rows_per_tile = n_rows // NS
    kp = k_chunks // n_phases
    n_streams = kp // s_chk
    cs = s_chk * c
    assert n_streams % nbuf == 0 and k_chunks % (n_phases * s_chk) == 0
    idx_shape = (n_streams, cs)
    mesh = plsc.VectorSubcoreMesh(core_axis_name="c", subcore_axis_name="s")
    scratch = [pltpu.VMEM(idx_shape, jnp.int32)]
    if gather:
        scratch.append(pltpu.VMEM(idx_shape, jnp.int32))
        scratch += [pltpu.VMEM((cs, d), jnp.float32) for _ in range(nbuf)]
        scratch += [pltpu.SemaphoreType.DMA for _ in range(2 * nbuf)]
    else:
        scratch.append(pltpu.VMEM((cs, d), jnp.float32))
        scratch += [pltpu.SemaphoreType.DMA for _ in range(nbuf)]
    scratch.append(pltpu.VMEM_SHARED((n_rows, d), jnp.float32))

    def body(*refs):
        if gather:
            table, srcr, dstr, out = refs[:4]
            dst_v, src_v = refs[4:6]
            rows = refs[6:6 + nbuf]
            gsem = refs[6 + nbuf:6 + 2 * nbuf]
            ssem = refs[6 + 2 * nbuf:6 + 3 * nbuf]
            acc = refs[6 + 3 * nbuf]
        else:
            dstr, out, dst_v, ones_v = refs[:4]
            ssem = refs[4:4 + nbuf]
            acc = refs[4 + nbuf]
        cid = lax.axis_index("c")
        sid = lax.axis_index("s")
        w = cid * NS + sid

        zbuf = rows[0] if gather else ones_v
        _fill_const(zbuf, cs, d, 0.0)
        base = sid * rows_per_tile
        off = 0
        while off < rows_per_tile:
            sz = min(cs, rows_per_tile - off)
            pltpu.sync_copy(zbuf.at[pl.ds(0, sz)],
                            acc.at[pl.ds(base + off, sz)])
            off += sz
        if not gather:
            _fill_const(ones_v, cs, d, 1.0)
        plsc.subcore_barrier()

        for ph in range(n_phases):
            p = w * n_phases + ph
            pltpu.sync_copy(dstr.at[p], dst_v)
            if gather:
                pltpu.sync_copy(srcr.at[p], src_v)
                for b in range(nbuf):
                    pltpu.async_copy(table.at[src_v.at[b]], rows[b], gsem[b])

                @pl.loop(0, n_streams // nbuf)
                def _(tt):
                    t = tt * nbuf
                    for b in range(nbuf):
                        j = t + b
                        pltpu.make_async_copy(
                            table.at[pl.ds(0, cs)], rows[b], gsem[b]).wait()
                        pltpu.async_copy(rows[b], acc.at[dst_v.at[j]],
                                         ssem[b], add=True)
                        pltpu.make_async_copy(
                            rows[b], acc.at[pl.ds(0, cs)], ssem[b]).wait()

                        @pl.when(j + nbuf < n_streams)
                        def _():
                            pltpu.async_copy(table.at[src_v.at[j + nbuf]],
                                             rows[b], gsem[b])
            else:
                for b in range(nbuf):
                    pltpu.async_copy(ones_v, acc.at[dst_v.at[b]],
                                     ssem[b], add=True)

                @pl.loop(0, n_streams // nbuf)
                def _(tt):
                    t = tt * nbuf
                    for b in range(nbuf):
                        j = t + b
                        pltpu.make_async_copy(
                            ones_v, acc.at[pl.ds(0, cs)], ssem[b]).wait()

                        @pl.when(j + nbuf < n_streams)
                        def _():
                            pltpu.async_copy(ones_v,
                                             acc.at[dst_v.at[j + nbuf]],
                                             ssem[b], add=True)

        plsc.subcore_barrier()
        off = 0
        while off < rows_per_tile:
            sz = min(cs, rows_per_tile - off)
            pltpu.sync_copy(acc.at[pl.ds(base + off, sz)],
                            out.at[pl.ds(cid * n_rows + base + off, sz)])
            off += sz

    return pl.kernel(
        body,
        out_type=jax.ShapeDtypeStruct((NC * n_rows, d), jnp.float32),
        mesh=mesh,
        scratch_types=scratch,
        compiler_params=pltpu.CompilerParams(use_tc_tiling_on_sc=tc_tiling),
    )


def kernel(x, edge_index, W1, b1, gamma, beta, Wskip, bskip, W2, b2):
    n, d_in = x.shape
    d_h = W1.shape[1]
    n_cls = W2.shape[1]
    e = edge_index.shape[1]
    n_rows = n + PAD_ROWS

    def _pad_edges(shape):
        pad = 1
        for s_ in shape:
            pad *= s_
        pad -= e
        ar = jnp.arange(pad, dtype=jnp.int32)
        s = jnp.concatenate([edge_index[0], (ar * 37) % n])
        t = jnp.concatenate([edge_index[1], n + ar % PAD_ROWS])
        return s.reshape(shape), t.reshape(shape)

    ka = -(-e // (NW * 512)) * 4
    sa, ta = _pad_edges((NW, ka // 4, 4 * C))
    kb = -(-e // (NW * 72 * 16)) * 16
    sb, tb = _pad_edges((NW * 4, kb // 4, 72))

    degp = _make_agg(n_rows, 16, ka, gather=False, nbuf=4, s_chk=4,
                     tc_tiling=False)(ta)
    degp = degp.reshape(NC, n_rows, 16)

    R = 1000
    grid = (n // R,)

    def _kpre(x_ref, w_ref, wsk_ref, bsk_ref, h_ref, xr_ref):
        xv = x_ref[...]
        h_ref[...] = jnp.dot(xv, w_ref[...],
                             preferred_element_type=jnp.float32)
        xr_ref[...] = jnp.dot(xv, wsk_ref[...],
                              preferred_element_type=jnp.float32) + bsk_ref[...]

    h, xr = pl.pallas_call(
        _kpre,
        grid=grid,
        in_specs=[
            pl.BlockSpec((R, d_in), lambda i: (i, 0)),
            pl.BlockSpec((d_in, d_h), lambda i: (0, 0)),
            pl.BlockSpec((d_in, d_h), lambda i: (0, 0)),
            pl.BlockSpec((1, d_h), lambda i: (0, 0)),
        ],
        out_specs=[pl.BlockSpec((R, d_h), lambda i: (i, 0)),
                   pl.BlockSpec((R, d_h), lambda i: (i, 0))],
        out_shape=[jax.ShapeDtypeStruct((n, d_h), jnp.float32),
                   jax.ShapeDtypeStruct((n, d_h), jnp.float32)],
    )(x, W1, Wskip, bskip.reshape(1, d_h))

    def _ka(h_ref, dg_ref, g_ref):
        deg = dg_ref[0, :, 0] + dg_ref[1, :, 0] + 1.0
        dis = lax.rsqrt(deg)
        g_ref[...] = dis[:, None] * h_ref[...]

    g = pl.pallas_call(
        _ka,
        grid=grid,
        in_specs=[
            pl.BlockSpec((R, d_h), lambda i: (i, 0)),
            pl.BlockSpec((NC, R, 16), lambda i: (0, i, 0)),
        ],
        out_specs=pl.BlockSpec((R, d_h), lambda i: (i, 0)),
        out_shape=jax.ShapeDtypeStruct((n, d_h), jnp.float32),
    )(h, degp)

    sp = _make_agg(n_rows, d_h, kb, gather=True, nbuf=4, c=72,
                   n_phases=4)(g, sb, tb)
    sp = sp.reshape(NC, n_rows, d_h)

    w2p = jnp.pad(W2, ((0, 0), (0, 16 - n_cls)))

    def _kb(sp_ref, g_ref, xr_ref, dg_ref, b1_ref,
            ga_ref, be_ref, w2_ref, out_ref):
        deg = dg_ref[0, :, 0] + dg_ref[1, :, 0] + 1.0
        dis = lax.rsqrt(deg)
        spv = sp_ref[...]
        h1 = dis[:, None] * (spv[0] + spv[1] + g_ref[...]) + b1_ref[...]
        mu = jnp.mean(h1, axis=1, keepdims=True)
        var = jnp.mean((h1 - mu) ** 2, axis=1, keepdims=True)
        ln = (h1 - mu) * lax.rsqrt(var + 1e-5) * ga_ref[...] + be_ref[...]
        z = jnp.maximum(0.8 * ln + 0.2 * xr_ref[...], 0.0)
        out_ref[...] = dis[:, None] * jnp.dot(
            z, w2_ref[...], preferred_element_type=jnp.float32)

    g2 = pl.pallas_call(
        _kb,
        grid=grid,
        in_specs=[
            pl.BlockSpec((NC, R, d_h), lambda i: (0, i, 0)),
            pl.BlockSpec((R, d_h), lambda i: (i, 0)),
            pl.BlockSpec((R, d_h), lambda i: (i, 0)),
            pl.BlockSpec((NC, R, 16), lambda i: (0, i, 0)),
            pl.BlockSpec((1, d_h), lambda i: (0, 0)),
            pl.BlockSpec((1, d_h), lambda i: (0, 0)),
            pl.BlockSpec((1, d_h), lambda i: (0, 0)),
            pl.BlockSpec((d_h, 16), lambda i: (0, 0)),
        ],
        out_specs=pl.BlockSpec((R, 16), lambda i: (i, 0)),
        out_shape=jax.ShapeDtypeStruct((n, 16), jnp.float32),
    )(sp, g, xr, degp, b1.reshape(1, d_h),
      gamma.reshape(1, d_h), beta.reshape(1, d_h), w2p)

    s2p = _make_agg(n_rows, 16, ka, gather=True, nbuf=4, s_chk=4,
                    tc_tiling=False)(g2, sa, ta)
    s2p = s2p.reshape(NC, n_rows, 16)

    def _kc(s2_ref, g2_ref, dg_ref, b2_ref, out_ref):
        deg = dg_ref[0, :, 0] + dg_ref[1, :, 0] + 1.0
        dis = lax.rsqrt(deg)
        s2v = s2_ref[...]
        tot = dis[:, None] * (s2v[0] + s2v[1] + g2_ref[...])
        out_ref[...] = tot[:, :8] + b2_ref[...]

    out = pl.pallas_call(
        _kc,
        grid=grid,
        in_specs=[
            pl.BlockSpec((NC, R, 16), lambda i: (0, i, 0)),
            pl.BlockSpec((R, 16), lambda i: (i, 0)),
            pl.BlockSpec((NC, R, 16), lambda i: (0, i, 0)),
            pl.BlockSpec((1, n_cls), lambda i: (0, 0)),
        ],
        out_specs=pl.BlockSpec((R, n_cls), lambda i: (i, 0)),
        out_shape=jax.ShapeDtypeStruct((n, n_cls), jnp.float32),
    )(s2p, g2, degp, b2.reshape(1, n_cls))

    return out


# speedup vs baseline: 1.0357x; 1.0357x over previous
import functools

import jax
import jax.numpy as jnp
from jax import lax
from jax.experimental import pallas as pl
from jax.experimental.pallas import tpu as pltpu
from jax.experimental.pallas import tpu_sc as plsc

NC = 2
NS = 16
NW = NC * NS
C = 128
PAD_ROWS = 112


def _fill_const(ref, rows, d, val):
    @pl.loop(0, rows)
    def _(r):
        for cc in range(d // 16):
            ref[r, pl.ds(cc * 16, 16)] = jnp.full((16,), val, jnp.float32)


def _make_agg(n_rows, d, k_chunks, gather, nbuf, c=C, s_chk=1, n_phases=1,
              tc_tiling=True):
    rows_per_tile = n_rows // NS
    kp = k_chunks // n_phases
    n_streams = kp // s_chk
    cs = s_chk * c
    assert n_streams % nbuf == 0 and k_chunks % (n_phases * s_chk) == 0
    idx_shape = (n_streams, cs)
    mesh = plsc.VectorSubcoreMesh(core_axis_name="c", subcore_axis_name="s")
    scratch = [pltpu.VMEM(idx_shape, jnp.int32)]
    if gather:
        scratch.append(pltpu.VMEM(idx_shape, jnp.int32))
        scratch += [pltpu.VMEM((cs, d), jnp.float32) for _ in range(nbuf)]
        scratch += [pltpu.SemaphoreType.DMA for _ in range(2 * nbuf)]
    else:
        scratch.append(pltpu.VMEM((cs, d), jnp.float32))
        scratch += [pltpu.SemaphoreType.DMA for _ in range(nbuf)]
    scratch.append(pltpu.VMEM_SHARED((n_rows, d), jnp.float32))

    def body(*refs):
        if gather:
            table, srcr, dstr, out = refs[:4]
            dst_v, src_v = refs[4:6]
            rows = refs[6:6 + nbuf]
            gsem = refs[6 + nbuf:6 + 2 * nbuf]
            ssem = refs[6 + 2 * nbuf:6 + 3 * nbuf]
            acc = refs[6 + 3 * nbuf]
        else:
            dstr, out, dst_v, ones_v = refs[:4]
            ssem = refs[4:4 + nbuf]
            acc = refs[4 + nbuf]
        cid = lax.axis_index("c")
        sid = lax.axis_index("s")
        w = cid * NS + sid

        zbuf = rows[0] if gather else ones_v
        _fill_const(zbuf, cs, d, 0.0)
        base = sid * rows_per_tile
        off = 0
        while off < rows_per_tile:
            sz = min(cs, rows_per_tile - off)
            pltpu.sync_copy(zbuf.at[pl.ds(0, sz)],
                            acc.at[pl.ds(base + off, sz)])
            off += sz
        if not gather:
            _fill_const(ones_v, cs, d, 1.0)
        plsc.subcore_barrier()

        for ph in range(n_phases):
            p = w * n_phases + ph
            pltpu.sync_copy(dstr.at[p], dst_v)
            if gather:
                pltpu.sync_copy(srcr.at[p], src_v)
                for b in range(nbuf):
                    pltpu.async_copy(table.at[src_v.at[b]], rows[b], gsem[b])

                @pl.loop(0, n_streams // nbuf)
                def _(tt):
                    t = tt * nbuf
                    for b in range(nbuf):
                        j = t + b
                        pltpu.make_async_copy(
                            table.at[pl.ds(0, cs)], rows[b], gsem[b]).wait()
                        pltpu.async_copy(rows[b], acc.at[dst_v.at[j]],
                                         ssem[b], add=True)
                        pltpu.make_async_copy(
                            rows[b], acc.at[pl.ds(0, cs)], ssem[b]).wait()

                        @pl.when(j + nbuf < n_streams)
                        def _():
                            pltpu.async_copy(table.at[src_v.at[j + nbuf]],
                                             rows[b], gsem[b])
            else:
                for b in range(nbuf):
                    pltpu.async_copy(ones_v, acc.at[dst_v.at[b]],
                                     ssem[b], add=True)

                @pl.loop(0, n_streams // nbuf)
                def _(tt):
                    t = tt * nbuf
                    for b in range(nbuf):
                        j = t + b
                        pltpu.make_async_copy(
                            ones_v, acc.at[pl.ds(0, cs)], ssem[b]).wait()

                        @pl.when(j + nbuf < n_streams)
                        def _():
                            pltpu.async_copy(ones_v,
                                             acc.at[dst_v.at[j + nbuf]],
                                             ssem[b], add=True)

        plsc.subcore_barrier()
        off = 0
        while off < rows_per_tile:
            sz = min(cs, rows_per_tile - off)
            pltpu.sync_copy(acc.at[pl.ds(base + off, sz)],
                            out.at[pl.ds(cid * n_rows + base + off, sz)])
            off += sz

    return pl.kernel(
        body,
        out_type=jax.ShapeDtypeStruct((NC * n_rows, d), jnp.float32),
        mesh=mesh,
        scratch_types=scratch,
        compiler_params=pltpu.CompilerParams(use_tc_tiling_on_sc=tc_tiling),
    )


def kernel(x, edge_index, W1, b1, gamma, beta, Wskip, bskip, W2, b2):
    n, d_in = x.shape
    d_h = W1.shape[1]
    n_cls = W2.shape[1]
    e = edge_index.shape[1]
    n_rows = n + PAD_ROWS

    def _pad_edges(shape):
        pad = 1
        for s_ in shape:
            pad *= s_
        pad -= e
        ar = jnp.arange(pad, dtype=jnp.int32)
        s = jnp.concatenate([edge_index[0], (ar * 37) % n])
        t = jnp.concatenate([edge_index[1], n + ar % PAD_ROWS])
        return s.reshape(shape), t.reshape(shape)

    ka = -(-e // (NW * 512)) * 4
    sa, ta = _pad_edges((NW, ka // 4, 4 * C))
    kb = -(-e // (NW * 88 * 6)) * 6
    sb, tb = _pad_edges((NW * 2, kb // 2, 88))

    degp = _make_agg(n_rows, 16, ka, gather=False, nbuf=4, s_chk=4,
                     tc_tiling=False)(ta)
    degp = degp.reshape(NC, n_rows, 16)

    R = 1000
    grid = (n // R,)

    def _kpre(x_ref, w_ref, wsk_ref, bsk_ref, h_ref, xr_ref):
        xv = x_ref[...]
        h_ref[...] = jnp.dot(xv, w_ref[...],
                             preferred_element_type=jnp.float32)
        xr_ref[...] = jnp.dot(xv, wsk_ref[...],
                              preferred_element_type=jnp.float32) + bsk_ref[...]

    h, xr = pl.pallas_call(
        _kpre,
        grid=grid,
        in_specs=[
            pl.BlockSpec((R, d_in), lambda i: (i, 0)),
            pl.BlockSpec((d_in, d_h), lambda i: (0, 0)),
            pl.BlockSpec((d_in, d_h), lambda i: (0, 0)),
            pl.BlockSpec((1, d_h), lambda i: (0, 0)),
        ],
        out_specs=[pl.BlockSpec((R, d_h), lambda i: (i, 0)),
                   pl.BlockSpec((R, d_h), lambda i: (i, 0))],
        out_shape=[jax.ShapeDtypeStruct((n, d_h), jnp.float32),
                   jax.ShapeDtypeStruct((n, d_h), jnp.float32)],
    )(x, W1, Wskip, bskip.reshape(1, d_h))

    def _ka(h_ref, dg_ref, g_ref):
        deg = dg_ref[0, :, 0] + dg_ref[1, :, 0] + 1.0
        dis = lax.rsqrt(deg)
        g_ref[...] = dis[:, None] * h_ref[...]

    g = pl.pallas_call(
        _ka,
        grid=grid,
        in_specs=[
            pl.BlockSpec((R, d_h), lambda i: (i, 0)),
            pl.BlockSpec((NC, R, 16), lambda i: (0, i, 0)),
        ],
        out_specs=pl.BlockSpec((R, d_h), lambda i: (i, 0)),
        out_shape=jax.ShapeDtypeStruct((n, d_h), jnp.float32),
    )(h, degp)

    sp = _make_agg(n_rows, d_h, kb, gather=True, nbuf=3, c=88,
                   n_phases=2)(g, sb, tb)
    sp = sp.reshape(NC, n_rows, d_h)

    w2p = jnp.pad(W2, ((0, 0), (0, 16 - n_cls)))

    def _kb(sp_ref, g_ref, xr_ref, dg_ref, b1_ref,
            ga_ref, be_ref, w2_ref, out_ref):
        deg = dg_ref[0, :, 0] + dg_ref[1, :, 0] + 1.0
        dis = lax.rsqrt(deg)
        spv = sp_ref[...]
        h1 = dis[:, None] * (spv[0] + spv[1] + g_ref[...]) + b1_ref[...]
        mu = jnp.mean(h1, axis=1, keepdims=True)
        var = jnp.mean((h1 - mu) ** 2, axis=1, keepdims=True)
        ln = (h1 - mu) * lax.rsqrt(var + 1e-5) * ga_ref[...] + be_ref[...]
        z = jnp.maximum(0.8 * ln + 0.2 * xr_ref[...], 0.0)
        out_ref[...] = dis[:, None] * jnp.dot(
            z, w2_ref[...], preferred_element_type=jnp.float32)

    g2 = pl.pallas_call(
        _kb,
        grid=grid,
        in_specs=[
            pl.BlockSpec((NC, R, d_h), lambda i: (0, i, 0)),
            pl.BlockSpec((R, d_h), lambda i: (i, 0)),
            pl.BlockSpec((R, d_h), lambda i: (i, 0)),
            pl.BlockSpec((NC, R, 16), lambda i: (0, i, 0)),
            pl.BlockSpec((1, d_h), lambda i: (0, 0)),
            pl.BlockSpec((1, d_h), lambda i: (0, 0)),
            pl.BlockSpec((1, d_h), lambda i: (0, 0)),
            pl.BlockSpec((d_h, 16), lambda i: (0, 0)),
        ],
        out_specs=pl.BlockSpec((R, 16), lambda i: (i, 0)),
        out_shape=jax.ShapeDtypeStruct((n, 16), jnp.float32),
    )(sp, g, xr, degp, b1.reshape(1, d_h),
      gamma.reshape(1, d_h), beta.reshape(1, d_h), w2p)

    s2p = _make_agg(n_rows, 16, ka, gather=True, nbuf=4, s_chk=4,
                    tc_tiling=False)(g2, sa, ta)
    s2p = s2p.reshape(NC, n_rows, 16)

    def _kc(s2_ref, g2_ref, dg_ref, b2_ref, out_ref):
        deg = dg_ref[0, :, 0] + dg_ref[1, :, 0] + 1.0
        dis = lax.rsqrt(deg)
        s2v = s2_ref[...]
        tot = dis[:, None] * (s2v[0] + s2v[1] + g2_ref[...])
        out_ref[...] = tot[:, :8] + b2_ref[...]

    out = pl.pallas_call(
        _kc,
        grid=grid,
        in_specs=[
            pl.BlockSpec((NC, R, 16), lambda i: (0, i, 0)),
            pl.BlockSpec((R, 16), lambda i: (i, 0)),
            pl.BlockSpec((NC, R, 16), lambda i: (0, i, 0)),
            pl.BlockSpec((1, n_cls), lambda i: (0, 0)),
        ],
        out_specs=pl.BlockSpec((R, n_cls), lambda i: (i, 0)),
        out_shape=jax.ShapeDtypeStruct((n, n_cls), jnp.float32),
    )(s2p, g2, degp, b2.reshape(1, n_cls))

    return out


# speedup vs baseline: 1.0462x; 1.0101x over previous
import functools

import jax
import jax.numpy as jnp
from jax import lax
from jax.experimental import pallas as pl
from jax.experimental.pallas import tpu as pltpu
from jax.experimental.pallas import tpu_sc as plsc

NC = 2
NS = 16
NW = NC * NS
C = 128
PAD_ROWS = 112


def _fill_const(ref, rows, d, val):
    @pl.loop(0, rows)
    def _(r):
        for cc in range(d // 16):
            ref[r, pl.ds(cc * 16, 16)] = jnp.full((16,), val, jnp.float32)


def _make_agg(n_rows, d, k_chunks, gather, nbuf, c=C, s_chk=1, n_phases=1,
              tc_tiling=True):
    rows_per_tile = n_rows // NS
    kp = k_chunks // n_phases
    n_streams = kp // s_chk
    cs = s_chk * c
    assert n_streams % nbuf == 0 and k_chunks % (n_phases * s_chk) == 0
    idx_shape = (n_streams, cs)
    mesh = plsc.VectorSubcoreMesh(core_axis_name="c", subcore_axis_name="s")
    scratch = [pltpu.VMEM(idx_shape, jnp.int32)]
    if gather:
        scratch.append(pltpu.VMEM(idx_shape, jnp.int32))
        scratch += [pltpu.VMEM((cs, d), jnp.float32) for _ in range(nbuf)]
        scratch += [pltpu.SemaphoreType.DMA for _ in range(2 * nbuf)]
    else:
        scratch.append(pltpu.VMEM((cs, d), jnp.float32))
        scratch += [pltpu.SemaphoreType.DMA for _ in range(nbuf)]
    scratch.append(pltpu.VMEM_SHARED((n_rows, d), jnp.float32))

    def body(*refs):
        if gather:
            table, srcr, dstr, out = refs[:4]
            dst_v, src_v = refs[4:6]
            rows = refs[6:6 + nbuf]
            gsem = refs[6 + nbuf:6 + 2 * nbuf]
            ssem = refs[6 + 2 * nbuf:6 + 3 * nbuf]
            acc = refs[6 + 3 * nbuf]
        else:
            dstr, out, dst_v, ones_v = refs[:4]
            ssem = refs[4:4 + nbuf]
            acc = refs[4 + nbuf]
        cid = lax.axis_index("c")
        sid = lax.axis_index("s")
        w = cid * NS + sid

        zbuf = rows[0] if gather else ones_v
        _fill_const(zbuf, cs, d, 0.0)
        base = sid * rows_per_tile
        off = 0
        while off < rows_per_tile:
            sz = min(cs, rows_per_tile - off)
            pltpu.sync_copy(zbuf.at[pl.ds(0, sz)],
                            acc.at[pl.ds(base + off, sz)])
            off += sz
        if not gather:
            _fill_const(ones_v, cs, d, 1.0)
        plsc.subcore_barrier()

        for ph in range(n_phases):
            p = w * n_phases + ph
            pltpu.sync_copy(dstr.at[p], dst_v)
            if gather:
                pltpu.sync_copy(srcr.at[p], src_v)
                for b in range(nbuf):
                    pltpu.async_copy(table.at[src_v.at[b]], rows[b], gsem[b])

                @pl.loop(0, n_streams // nbuf)
                def _(tt):
                    t = tt * nbuf
                    for b in range(nbuf):
                        j = t + b
                        pltpu.make_async_copy(
                            table.at[pl.ds(0, cs)], rows[b], gsem[b]).wait()
                        pltpu.async_copy(rows[b], acc.at[dst_v.at[j]],
                                         ssem[b], add=True)
                        pltpu.make_async_copy(
                            rows[b], acc.at[pl.ds(0, cs)], ssem[b]).wait()

                        @pl.when(j + nbuf < n_streams)
                        def _():
                            pltpu.async_copy(table.at[src_v.at[j + nbuf]],
                                             rows[b], gsem[b])
            else:
                for b in range(nbuf):
                    pltpu.async_copy(ones_v, acc.at[dst_v.at[b]],
                                     ssem[b], add=True)

                @pl.loop(0, n_streams // nbuf)
                def _(tt):
                    t = tt * nbuf
                    for b in range(nbuf):
                        j = t + b
                        pltpu.make_async_copy(
                            ones_v, acc.at[pl.ds(0, cs)], ssem[b]).wait()

                        @pl.when(j + nbuf < n_streams)
                        def _():
                            pltpu.async_copy(ones_v,
                                             acc.at[dst_v.at[j + nbuf]],
                                             ssem[b], add=True)

        plsc.subcore_barrier()
        off = 0
        while off < rows_per_tile:
            sz = min(cs, rows_per_tile - off)
            pltpu.sync_copy(acc.at[pl.ds(base + off, sz)],
                            out.at[pl.ds(cid * n_rows + base + off, sz)])
            off += sz

    return pl.kernel(
        body,
        out_type=jax.ShapeDtypeStruct((NC * n_rows, d), jnp.float32),
        mesh=mesh,
        scratch_types=scratch,
        compiler_params=pltpu.CompilerParams(use_tc_tiling_on_sc=tc_tiling),
    )


def kernel(x, edge_index, W1, b1, gamma, beta, Wskip, bskip, W2, b2):
    n, d_in = x.shape
    d_h = W1.shape[1]
    n_cls = W2.shape[1]
    e = edge_index.shape[1]
    n_rows = n + PAD_ROWS

    def _pad_edges(shape):
        pad = 1
        for s_ in shape:
            pad *= s_
        pad -= e
        ar = jnp.arange(pad, dtype=jnp.int32)
        s = jnp.concatenate([edge_index[0], (ar * 37) % n])
        t = jnp.concatenate([edge_index[1], n + ar % PAD_ROWS])
        return s.reshape(shape), t.reshape(shape)

    ka = -(-e // (NW * 512)) * 4
    sa, ta = _pad_edges((NW, ka // 4, 4 * C))
    kb = -(-e // (NW * 88 * 6)) * 6
    sb, tb = _pad_edges((NW * 2, kb // 2, 88))

    degp = _make_agg(n_rows, 16, ka, gather=False, nbuf=4, s_chk=4,
                     tc_tiling=False)(ta)
    degp = degp.reshape(NC, n_rows, 16)

    R = 1000
    grid = (n // R,)

    def _kpre(x_ref, w_ref, wsk_ref, bsk_ref, h_ref, xr_ref):
        xv = x_ref[...]
        h_ref[...] = jnp.dot(xv, w_ref[...],
                             preferred_element_type=jnp.float32)
        xr_ref[...] = jnp.dot(xv, wsk_ref[...],
                              preferred_element_type=jnp.float32) + bsk_ref[...]

    h, xr = pl.pallas_call(
        _kpre,
        grid=grid,
        in_specs=[
            pl.BlockSpec((R, d_in), lambda i: (i, 0)),
            pl.BlockSpec((d_in, d_h), lambda i: (0, 0)),
            pl.BlockSpec((d_in, d_h), lambda i: (0, 0)),
            pl.BlockSpec((1, d_h), lambda i: (0, 0)),
        ],
        out_specs=[pl.BlockSpec((R, d_h), lambda i: (i, 0)),
                   pl.BlockSpec((R, d_h), lambda i: (i, 0))],
        out_shape=[jax.ShapeDtypeStruct((n, d_h), jnp.float32),
                   jax.ShapeDtypeStruct((n, d_h), jnp.float32)],
    )(x, W1, Wskip, bskip.reshape(1, d_h))

    def _ka(h_ref, dg_ref, g_ref):
        deg = dg_ref[0, :, 0] + dg_ref[1, :, 0] + 1.0
        dis = lax.rsqrt(deg)
        g_ref[...] = dis[:, None] * h_ref[...]

    g = pl.pallas_call(
        _ka,
        grid=grid,
        in_specs=[
            pl.BlockSpec((R, d_h), lambda i: (i, 0)),
            pl.BlockSpec((NC, R, 16), lambda i: (0, i, 0)),
        ],
        out_specs=pl.BlockSpec((R, d_h), lambda i: (i, 0)),
        out_shape=jax.ShapeDtypeStruct((n, d_h), jnp.float32),
    )(h, degp)

    sp = _make_agg(n_rows, d_h, kb, gather=True, nbuf=3, c=88,
                   n_phases=2, tc_tiling=False)(g, sb, tb)
    sp = sp.reshape(NC, n_rows, d_h)

    w2p = jnp.pad(W2, ((0, 0), (0, 16 - n_cls)))

    def _kb(sp_ref, g_ref, xr_ref, dg_ref, b1_ref,
            ga_ref, be_ref, w2_ref, out_ref):
        deg = dg_ref[0, :, 0] + dg_ref[1, :, 0] + 1.0
        dis = lax.rsqrt(deg)
        spv = sp_ref[...]
        h1 = dis[:, None] * (spv[0] + spv[1] + g_ref[...]) + b1_ref[...]
        mu = jnp.mean(h1, axis=1, keepdims=True)
        var = jnp.mean((h1 - mu) ** 2, axis=1, keepdims=True)
        ln = (h1 - mu) * lax.rsqrt(var + 1e-5) * ga_ref[...] + be_ref[...]
        z = jnp.maximum(0.8 * ln + 0.2 * xr_ref[...], 0.0)
        out_ref[...] = dis[:, None] * jnp.dot(
            z, w2_ref[...], preferred_element_type=jnp.float32)

    g2 = pl.pallas_call(
        _kb,
        grid=grid,
        in_specs=[
            pl.BlockSpec((NC, R, d_h), lambda i: (0, i, 0)),
            pl.BlockSpec((R, d_h), lambda i: (i, 0)),
            pl.BlockSpec((R, d_h), lambda i: (i, 0)),
            pl.BlockSpec((NC, R, 16), lambda i: (0, i, 0)),
            pl.BlockSpec((1, d_h), lambda i: (0, 0)),
            pl.BlockSpec((1, d_h), lambda i: (0, 0)),
            pl.BlockSpec((1, d_h), lambda i: (0, 0)),
            pl.BlockSpec((d_h, 16), lambda i: (0, 0)),
        ],
        out_specs=pl.BlockSpec((R, 16), lambda i: (i, 0)),
        out_shape=jax.ShapeDtypeStruct((n, 16), jnp.float32),
    )(sp, g, xr, degp, b1.reshape(1, d_h),
      gamma.reshape(1, d_h), beta.reshape(1, d_h), w2p)

    s2p = _make_agg(n_rows, 16, ka, gather=True, nbuf=4, s_chk=4,
                    tc_tiling=False)(g2, sa, ta)
    s2p = s2p.reshape(NC, n_rows, 16)

    def _kc(s2_ref, g2_ref, dg_ref, b2_ref, out_ref):
        deg = dg_ref[0, :, 0] + dg_ref[1, :, 0] + 1.0
        dis = lax.rsqrt(deg)
        s2v = s2_ref[...]
        tot = dis[:, None] * (s2v[0] + s2v[1] + g2_ref[...])
        out_ref[...] = tot[:, :8] + b2_ref[...]

    out = pl.pallas_call(
        _kc,
        grid=grid,
        in_specs=[
            pl.BlockSpec((NC, R, 16), lambda i: (0, i, 0)),
            pl.BlockSpec((R, 16), lambda i: (i, 0)),
            pl.BlockSpec((NC, R, 16), lambda i: (0, i, 0)),
            pl.BlockSpec((1, n_cls), lambda i: (0, 0)),
        ],
        out_specs=pl.BlockSpec((R, n_cls), lambda i: (i, 0)),
        out_shape=jax.ShapeDtypeStruct((n, n_cls), jnp.float32),
    )(s2p, g2, degp, b2.reshape(1, n_cls))

    return out
